# baseline (speedup 1.0000x reference)
import functools

import jax
import jax.numpy as jnp
from jax import lax
from jax.experimental import pallas as pl
from jax.experimental.pallas import tpu as pltpu
from jax.experimental.pallas import tpu_sc as plsc

B = 16384
D_OUT = 96
COLS = (0, 32, 48, 64, 80)
DIMS = (32, 16, 16, 16, 16)

_info = plsc.get_sparse_core_info()
NC, NS = _info.num_cores, _info.num_subcores
NW = NC * NS
BPW = B // NW
CH = 128
NCH = BPW // CH

_mesh = plsc.VectorSubcoreMesh(core_axis_name="c", subcore_axis_name="s")


@functools.partial(
    pl.kernel,
    mesh=_mesh,
    out_type=jax.ShapeDtypeStruct((B, D_OUT), jnp.float32),
    scratch_types=[
        pltpu.VMEM((5, NCH, CH), jnp.int32),
        pltpu.VMEM((BPW, 32), jnp.float32),
        pltpu.VMEM((BPW, 16), jnp.float32),
        pltpu.VMEM((BPW, 16), jnp.float32),
        pltpu.VMEM((BPW, 16), jnp.float32),
        pltpu.VMEM((BPW, 16), jnp.float32),
        pltpu.SemaphoreType.DMA,
    ],
    compiler_params=pltpu.CompilerParams(use_tc_tiling_on_sc=False),
)
def _gather_concat(acc_i, area_i, hour_i, wk_i, geo_i,
                   acc_t, area_t, hour_t, wk_t, geo_t,
                   out,
                   idx_v, acc_v, area_v, hour_v, wk_v, geo_v, sem):
    wid = lax.axis_index("s") * NC + lax.axis_index("c")
    row0 = wid * NCH
    base = wid * BPW

    idx_refs = (acc_i, area_i, hour_i, wk_i, geo_i)
    tabs = (acc_t, area_t, hour_t, wk_t, geo_t)
    bufs = (acc_v, area_v, hour_v, wk_v, geo_v)

    for f in range(5):
        pltpu.sync_copy(idx_refs[f].at[pl.ds(row0, NCH)], idx_v.at[f])

    copies = []
    for f in range(5):
        for j in range(NCH):
            copies.append(
                pltpu.async_copy(tabs[f].at[idx_v.at[f, j]],
                                 bufs[f].at[pl.ds(j * CH, CH)], sem))
    for c in copies:
        c.wait()

    for f in range(5):
        pltpu.sync_copy(bufs[f],
                        out.at[pl.ds(base, BPW), pl.ds(COLS[f], DIMS[f])])


def kernel(account_id, delivery_area_id, order_hour, order_weekday, geohash6,
           account_table, area_table, hour_table, weekday_table, geohash_table):
    def prep(i):
        return jnp.reshape(i.astype(jnp.int32), (B // CH, CH))

    return _gather_concat(
        prep(account_id), prep(delivery_area_id), prep(order_hour),
        prep(order_weekday), prep(geohash6),
        account_table, area_table, hour_table, weekday_table, geohash_table)
